# baseline (speedup 1.0000x reference)
import functools

import jax
import jax.numpy as jnp
from jax import lax
from jax.experimental import pallas as pl
from jax.experimental.pallas import tpu as pltpu
from jax.experimental.pallas import tpu_sc as plsc

VOCAB = 1000000
BATCH = 16384
FIELDS = 26

NUM_WORKERS = 32
LANES = 16
BPW = BATCH // NUM_WORKERS
IPW = BPW * FIELDS

W_SLICE = 62592
W_LAST = VOCAB - 15 * W_SLICE

GROUPS = ((0, 7), (7, 13), (13, 20), (20, 26))


def _sc_body(fid_hbm, fval_hbm, w_hbm, bias_hbm, out_hbm,
             idx_v, idx2_v, emb_v, val_v, out_v, bias_v, w_sh, sem, wsem):
    c = lax.axis_index("c")
    s = lax.axis_index("s")
    wid = s * 2 + c
    base_b = wid * BPW

    w_off = s * W_SLICE

    @pl.when(s < 15)
    def _():
        pltpu.make_async_copy(
            w_hbm.at[:, pl.ds(w_off, W_SLICE)],
            w_sh.at[:, pl.ds(w_off, W_SLICE)], wsem).start()

    @pl.when(s == 15)
    def _():
        pltpu.make_async_copy(
            w_hbm.at[:, pl.ds(15 * W_SLICE, W_LAST)],
            w_sh.at[:, pl.ds(15 * W_SLICE, W_LAST)], wsem).start()

    col = pl.multiple_of(base_b, BPW)
    for tr, nr in ((0, 8), (8, 8), (16, 8), (24, 2)):
        pltpu.make_async_copy(
            fid_hbm.at[pl.ds(tr, nr), pl.ds(col, BPW)],
            idx2_v.at[pl.ds(tr, nr), :], sem).start()
    for tr, nr in ((0, 8), (8, 8), (16, 8), (24, 2)):
        pltpu.make_async_copy(
            fval_hbm.at[pl.ds(tr, nr), pl.ds(col, BPW)],
            val_v.at[pl.ds(tr, nr), :], sem).start()
    pltpu.sync_copy(bias_hbm, bias_v)
    for tr, nr in ((0, 8), (8, 8), (16, 8), (24, 2)):
        pltpu.make_async_copy(
            fid_hbm.at[pl.ds(tr, nr), pl.ds(col, BPW)],
            idx2_v.at[pl.ds(tr, nr), :], sem).wait()

    def repack(cc, carry):
        off = cc * LANES
        for j in range(FIELDS):
            idx_v[pl.ds(j * BPW + off, LANES)] = idx2_v[j, pl.ds(off, LANES)]
        return carry

    lax.fori_loop(0, BPW // LANES, repack, 0)

    @pl.when(s < 15)
    def _():
        pltpu.make_async_copy(
            w_hbm.at[:, pl.ds(w_off, W_SLICE)],
            w_sh.at[:, pl.ds(w_off, W_SLICE)], wsem).wait()

    @pl.when(s == 15)
    def _():
        pltpu.make_async_copy(
            w_hbm.at[:, pl.ds(15 * W_SLICE, W_LAST)],
            w_sh.at[:, pl.ds(15 * W_SLICE, W_LAST)], wsem).wait()

    plsc.subcore_barrier()

    gathers = []
    for lo, hi in GROUPS:
        n = (hi - lo) * BPW
        g = pltpu.make_async_copy(
            w_sh.at[0].at[idx_v.at[pl.ds(lo * BPW, n)]],
            emb_v.at[pl.ds(lo * BPW, n)], sem)
        g.start()
        gathers.append(g)

    for tr, nr in ((0, 8), (8, 8), (16, 8), (24, 2)):
        pltpu.make_async_copy(
            fval_hbm.at[pl.ds(tr, nr), pl.ds(col, BPW)],
            val_v.at[pl.ds(tr, nr), :], sem).wait()

    zero = jnp.zeros((LANES,), jnp.int32)
    bias = plsc.load_gather(bias_v, [zero])

    for gi, (lo, hi) in enumerate(GROUPS):
        gathers[gi].wait()

        def group_body(cc, carry, lo=lo, hi=hi, first=(gi == 0)):
            off = cc * LANES
            acc = bias if first else out_v[pl.ds(off, LANES)]
            for j in range(lo, hi):
                e = emb_v[pl.ds(j * BPW + off, LANES)]
                v = val_v[j, pl.ds(off, LANES)]
                acc = acc + e * v
            out_v[pl.ds(off, LANES)] = acc
            return carry

        lax.fori_loop(0, BPW // LANES, group_body, 0)

    pltpu.sync_copy(out_v, out_hbm.at[pl.ds(base_b, BPW)])


_sc_kernel = functools.partial(
    pl.kernel,
    mesh=plsc.VectorSubcoreMesh(core_axis_name="c", subcore_axis_name="s"),
    out_type=jax.ShapeDtypeStruct((BATCH,), jnp.float32),
    scratch_types=[
        pltpu.VMEM((IPW,), jnp.int32),
        pltpu.VMEM((FIELDS, BPW), jnp.int32),
        pltpu.VMEM((IPW,), jnp.float32),
        pltpu.VMEM((FIELDS, BPW), jnp.float32),
        pltpu.VMEM((BPW,), jnp.float32),
        pltpu.VMEM((1,), jnp.float32),
        pltpu.VMEM_SHARED((1, VOCAB), jnp.float32),
        pltpu.SemaphoreType.DMA,
        pltpu.SemaphoreType.DMA,
    ],
    compiler_params=pltpu.CompilerParams(needs_layout_passes=False),
)(_sc_body)


@jax.jit
def kernel(feature_id, feature_val, W, bias):
    fid = feature_id.astype(jnp.int32).T
    return _sc_kernel(fid, feature_val.T, W.T, bias)


# speedup vs baseline: 1.0162x; 1.0162x over previous
import functools

import jax
import jax.numpy as jnp
from jax import lax
from jax.experimental import pallas as pl
from jax.experimental.pallas import tpu as pltpu
from jax.experimental.pallas import tpu_sc as plsc

VOCAB = 1000000
BATCH = 16384
FIELDS = 26

NUM_WORKERS = 32
LANES = 16
BPW = BATCH // NUM_WORKERS
IPW = BPW * FIELDS

W_SLICE = 62592
W_LAST = VOCAB - 15 * W_SLICE

GROUPS = ((0, 7), (7, 13), (13, 20), (20, 26))


def _sc_body(fid_hbm, fval_hbm, w_hbm, bias_hbm, out_hbm,
             idx_v, idx2_v, emb_v, val_v, out_v, bias_v, w_sh, sem, wsem):
    c = lax.axis_index("c")
    s = lax.axis_index("s")
    wid = s * 2 + c
    base_b = wid * BPW

    w_off = s * W_SLICE

    @pl.when(s < 15)
    def _():
        pltpu.make_async_copy(
            w_hbm.at[:, pl.ds(w_off, W_SLICE)],
            w_sh.at[:, pl.ds(w_off, W_SLICE)], wsem).start()

    @pl.when(s == 15)
    def _():
        pltpu.make_async_copy(
            w_hbm.at[:, pl.ds(15 * W_SLICE, W_LAST)],
            w_sh.at[:, pl.ds(15 * W_SLICE, W_LAST)], wsem).start()

    col = pl.multiple_of(base_b, BPW)
    for tr, nr in ((0, 8), (8, 8), (16, 8), (24, 2)):
        pltpu.make_async_copy(
            fid_hbm.at[pl.ds(tr, nr), pl.ds(col, BPW)],
            idx2_v.at[pl.ds(tr, nr), :], sem).start()
    for tr, nr in ((0, 8), (8, 8), (16, 8), (24, 2)):
        pltpu.make_async_copy(
            fval_hbm.at[pl.ds(tr, nr), pl.ds(col, BPW)],
            val_v.at[pl.ds(tr, nr), :], sem).start()
    pltpu.sync_copy(bias_hbm, bias_v)
    for tr, nr in ((0, 8), (8, 8), (16, 8), (24, 2)):
        pltpu.make_async_copy(
            fid_hbm.at[pl.ds(tr, nr), pl.ds(col, BPW)],
            idx2_v.at[pl.ds(tr, nr), :], sem).wait()

    def repack_group(lo, hi):
        def repack(cc, carry):
            off = cc * LANES
            for j in range(lo, hi):
                idx_v[pl.ds(j * BPW + off, LANES)] = idx2_v[j, pl.ds(off, LANES)]
            return carry

        lax.fori_loop(0, BPW // LANES, repack, 0)

    repack_group(*GROUPS[0])

    @pl.when(s < 15)
    def _():
        pltpu.make_async_copy(
            w_hbm.at[:, pl.ds(w_off, W_SLICE)],
            w_sh.at[:, pl.ds(w_off, W_SLICE)], wsem).wait()

    @pl.when(s == 15)
    def _():
        pltpu.make_async_copy(
            w_hbm.at[:, pl.ds(15 * W_SLICE, W_LAST)],
            w_sh.at[:, pl.ds(15 * W_SLICE, W_LAST)], wsem).wait()

    plsc.subcore_barrier()

    gathers = []
    for gi, (lo, hi) in enumerate(GROUPS):
        n = (hi - lo) * BPW
        g = pltpu.make_async_copy(
            w_sh.at[0].at[idx_v.at[pl.ds(lo * BPW, n)]],
            emb_v.at[pl.ds(lo * BPW, n)], sem)
        g.start()
        gathers.append(g)
        if gi + 1 < len(GROUPS):
            repack_group(*GROUPS[gi + 1])

    for tr, nr in ((0, 8), (8, 8), (16, 8), (24, 2)):
        pltpu.make_async_copy(
            fval_hbm.at[pl.ds(tr, nr), pl.ds(col, BPW)],
            val_v.at[pl.ds(tr, nr), :], sem).wait()

    zero = jnp.zeros((LANES,), jnp.int32)
    bias = plsc.load_gather(bias_v, [zero])

    for gi, (lo, hi) in enumerate(GROUPS):
        gathers[gi].wait()

        def group_body(cc, carry, lo=lo, hi=hi, first=(gi == 0)):
            off = cc * LANES
            acc = bias if first else out_v[pl.ds(off, LANES)]
            for j in range(lo, hi):
                e = emb_v[pl.ds(j * BPW + off, LANES)]
                v = val_v[j, pl.ds(off, LANES)]
                acc = acc + e * v
            out_v[pl.ds(off, LANES)] = acc
            return carry

        lax.fori_loop(0, BPW // LANES, group_body, 0)

    pltpu.sync_copy(out_v, out_hbm.at[pl.ds(base_b, BPW)])


_sc_kernel = functools.partial(
    pl.kernel,
    mesh=plsc.VectorSubcoreMesh(core_axis_name="c", subcore_axis_name="s"),
    out_type=jax.ShapeDtypeStruct((BATCH,), jnp.float32),
    scratch_types=[
        pltpu.VMEM((IPW,), jnp.int32),
        pltpu.VMEM((FIELDS, BPW), jnp.int32),
        pltpu.VMEM((IPW,), jnp.float32),
        pltpu.VMEM((FIELDS, BPW), jnp.float32),
        pltpu.VMEM((BPW,), jnp.float32),
        pltpu.VMEM((1,), jnp.float32),
        pltpu.VMEM_SHARED((1, VOCAB), jnp.float32),
        pltpu.SemaphoreType.DMA,
        pltpu.SemaphoreType.DMA,
    ],
    compiler_params=pltpu.CompilerParams(needs_layout_passes=False),
)(_sc_body)


@jax.jit
def kernel(feature_id, feature_val, W, bias):
    fid = feature_id.astype(jnp.int32).T
    return _sc_kernel(fid, feature_val.T, W.T, bias)


# speedup vs baseline: 1.0292x; 1.0128x over previous
import functools

import jax
import jax.numpy as jnp
from jax import lax
from jax.experimental import pallas as pl
from jax.experimental.pallas import tpu as pltpu
from jax.experimental.pallas import tpu_sc as plsc

VOCAB = 1000000
BATCH = 16384
FIELDS = 26

NUM_WORKERS = 32
LANES = 16
BPW = BATCH // NUM_WORKERS
IPW = BPW * FIELDS

W_SLICE = 62592
W_LAST = VOCAB - 15 * W_SLICE

GROUPS = ((0, 4), (4, 11), (11, 19), (19, 26))


def _sc_body(fid_hbm, fval_hbm, w_hbm, bias_hbm, out_hbm,
             idx_v, emb_v, val_v, out_v, bias_v, w_sh, sem, wsem):
    c = lax.axis_index("c")
    s = lax.axis_index("s")
    wid = s * 2 + c
    base_b = wid * BPW

    w_off = s * W_SLICE

    @pl.when(s < 15)
    def _():
        pltpu.make_async_copy(
            w_hbm.at[:, pl.ds(w_off, W_SLICE)],
            w_sh.at[:, pl.ds(w_off, W_SLICE)], wsem).start()

    @pl.when(s == 15)
    def _():
        pltpu.make_async_copy(
            w_hbm.at[:, pl.ds(15 * W_SLICE, W_LAST)],
            w_sh.at[:, pl.ds(15 * W_SLICE, W_LAST)], wsem).start()

    def stage(j, carry):
        pltpu.make_async_copy(
            fid_hbm.at[pl.ds(j * BATCH + base_b, BPW)],
            idx_v.at[pl.ds(j * BPW, BPW)], sem).start()
        return carry

    lax.fori_loop(0, FIELDS, stage, 0)
    col = pl.multiple_of(base_b, BPW)
    for tr, nr in ((0, 8), (8, 8), (16, 8), (24, 2)):
        pltpu.make_async_copy(
            fval_hbm.at[pl.ds(tr, nr), pl.ds(col, BPW)],
            val_v.at[pl.ds(tr, nr), :], sem).start()
    pltpu.sync_copy(bias_hbm, bias_v)

    def drain_idx(j, carry):
        pltpu.make_async_copy(
            fid_hbm.at[pl.ds(j * BATCH + base_b, BPW)],
            idx_v.at[pl.ds(j * BPW, BPW)], sem).wait()
        return carry

    lax.fori_loop(0, FIELDS, drain_idx, 0)

    @pl.when(s < 15)
    def _():
        pltpu.make_async_copy(
            w_hbm.at[:, pl.ds(w_off, W_SLICE)],
            w_sh.at[:, pl.ds(w_off, W_SLICE)], wsem).wait()

    @pl.when(s == 15)
    def _():
        pltpu.make_async_copy(
            w_hbm.at[:, pl.ds(15 * W_SLICE, W_LAST)],
            w_sh.at[:, pl.ds(15 * W_SLICE, W_LAST)], wsem).wait()

    plsc.subcore_barrier()

    gathers = []
    for lo, hi in GROUPS:
        n = (hi - lo) * BPW
        g = pltpu.make_async_copy(
            w_sh.at[0].at[idx_v.at[pl.ds(lo * BPW, n)]],
            emb_v.at[pl.ds(lo * BPW, n)], sem)
        g.start()
        gathers.append(g)

    for tr, nr in ((0, 8), (8, 8), (16, 8), (24, 2)):
        pltpu.make_async_copy(
            fval_hbm.at[pl.ds(tr, nr), pl.ds(col, BPW)],
            val_v.at[pl.ds(tr, nr), :], sem).wait()

    zero = jnp.zeros((LANES,), jnp.int32)
    bias = plsc.load_gather(bias_v, [zero])

    for gi, (lo, hi) in enumerate(GROUPS):
        gathers[gi].wait()

        def group_body(cc, carry, lo=lo, hi=hi, first=(gi == 0)):
            off = cc * LANES
            acc = bias if first else out_v[pl.ds(off, LANES)]
            for j in range(lo, hi):
                e = emb_v[pl.ds(j * BPW + off, LANES)]
                v = val_v[j, pl.ds(off, LANES)]
                acc = acc + e * v
            out_v[pl.ds(off, LANES)] = acc
            return carry

        lax.fori_loop(0, BPW // LANES, group_body, 0)

    pltpu.sync_copy(out_v, out_hbm.at[pl.ds(base_b, BPW)])


_sc_kernel = functools.partial(
    pl.kernel,
    mesh=plsc.VectorSubcoreMesh(core_axis_name="c", subcore_axis_name="s"),
    out_type=jax.ShapeDtypeStruct((BATCH,), jnp.float32),
    scratch_types=[
        pltpu.VMEM((IPW,), jnp.int32),
        pltpu.VMEM((IPW,), jnp.float32),
        pltpu.VMEM((FIELDS, BPW), jnp.float32),
        pltpu.VMEM((BPW,), jnp.float32),
        pltpu.VMEM((1,), jnp.float32),
        pltpu.VMEM_SHARED((1, VOCAB), jnp.float32),
        pltpu.SemaphoreType.DMA,
        pltpu.SemaphoreType.DMA,
    ],
    compiler_params=pltpu.CompilerParams(needs_layout_passes=False),
)(_sc_body)


@jax.jit
def kernel(feature_id, feature_val, W, bias):
    fid = feature_id.astype(jnp.int32).T.reshape(-1)
    return _sc_kernel(fid, feature_val.T, W.T, bias)


# speedup vs baseline: 1.0319x; 1.0026x over previous
import functools

import jax
import jax.numpy as jnp
from jax import lax
from jax.experimental import pallas as pl
from jax.experimental.pallas import tpu as pltpu
from jax.experimental.pallas import tpu_sc as plsc

VOCAB = 1000000
BATCH = 16384
FIELDS = 26

NUM_WORKERS = 32
LANES = 16
BPW = BATCH // NUM_WORKERS
IPW = BPW * FIELDS

W_SLICE = 62592
W_LAST = VOCAB - 15 * W_SLICE

GROUPS = ((0, 7), (7, 13), (13, 20), (20, 26))


def _sc_body(fid_hbm, fval_hbm, w_hbm, bias_hbm, out_hbm,
             idx_v, emb_v, val_v, out_v, bias_v, w_sh, sem, wsem):
    c = lax.axis_index("c")
    s = lax.axis_index("s")
    wid = s * 2 + c
    base_b = wid * BPW

    w_off = s * W_SLICE

    @pl.when(s < 15)
    def _():
        pltpu.make_async_copy(
            w_hbm.at[:, pl.ds(w_off, W_SLICE)],
            w_sh.at[:, pl.ds(w_off, W_SLICE)], wsem).start()

    @pl.when(s == 15)
    def _():
        pltpu.make_async_copy(
            w_hbm.at[:, pl.ds(15 * W_SLICE, W_LAST)],
            w_sh.at[:, pl.ds(15 * W_SLICE, W_LAST)], wsem).start()

    def stage(j, carry):
        pltpu.make_async_copy(
            fid_hbm.at[pl.ds(j * BATCH + base_b, BPW)],
            idx_v.at[pl.ds(j * BPW, BPW)], sem).start()
        return carry

    lax.fori_loop(0, FIELDS, stage, 0)
    col = pl.multiple_of(base_b, BPW)
    for tr, nr in ((0, 8), (8, 8), (16, 8), (24, 2)):
        pltpu.make_async_copy(
            fval_hbm.at[pl.ds(tr, nr), pl.ds(col, BPW)],
            val_v.at[pl.ds(tr, nr), :], sem).start()
    pltpu.sync_copy(bias_hbm, bias_v)

    def drain_idx(j, carry):
        pltpu.make_async_copy(
            fid_hbm.at[pl.ds(j * BATCH + base_b, BPW)],
            idx_v.at[pl.ds(j * BPW, BPW)], sem).wait()
        return carry

    lax.fori_loop(0, FIELDS, drain_idx, 0)

    @pl.when(s < 15)
    def _():
        pltpu.make_async_copy(
            w_hbm.at[:, pl.ds(w_off, W_SLICE)],
            w_sh.at[:, pl.ds(w_off, W_SLICE)], wsem).wait()

    @pl.when(s == 15)
    def _():
        pltpu.make_async_copy(
            w_hbm.at[:, pl.ds(15 * W_SLICE, W_LAST)],
            w_sh.at[:, pl.ds(15 * W_SLICE, W_LAST)], wsem).wait()

    plsc.subcore_barrier()

    gathers = []
    for lo, hi in GROUPS:
        n = (hi - lo) * BPW
        g = pltpu.make_async_copy(
            w_sh.at[0].at[idx_v.at[pl.ds(lo * BPW, n)]],
            emb_v.at[pl.ds(lo * BPW, n)], sem)
        g.start()
        gathers.append(g)

    for tr, nr in ((0, 8), (8, 8), (16, 8), (24, 2)):
        pltpu.make_async_copy(
            fval_hbm.at[pl.ds(tr, nr), pl.ds(col, BPW)],
            val_v.at[pl.ds(tr, nr), :], sem).wait()

    zero = jnp.zeros((LANES,), jnp.int32)
    bias = plsc.load_gather(bias_v, [zero])

    for gi, (lo, hi) in enumerate(GROUPS):
        gathers[gi].wait()

        def group_body(cc, carry, lo=lo, hi=hi, first=(gi == 0)):
            off = cc * LANES
            acc = bias if first else out_v[pl.ds(off, LANES)]
            for j in range(lo, hi):
                e = emb_v[pl.ds(j * BPW + off, LANES)]
                v = val_v[j, pl.ds(off, LANES)]
                acc = acc + e * v
            out_v[pl.ds(off, LANES)] = acc
            return carry

        lax.fori_loop(0, BPW // LANES, group_body, 0)

    pltpu.sync_copy(out_v, out_hbm.at[pl.ds(base_b, BPW)])


_sc_kernel = functools.partial(
    pl.kernel,
    mesh=plsc.VectorSubcoreMesh(core_axis_name="c", subcore_axis_name="s"),
    out_type=jax.ShapeDtypeStruct((BATCH,), jnp.float32),
    scratch_types=[
        pltpu.VMEM((IPW,), jnp.int32),
        pltpu.VMEM((IPW,), jnp.float32),
        pltpu.VMEM((FIELDS, BPW), jnp.float32),
        pltpu.VMEM((BPW,), jnp.float32),
        pltpu.VMEM((1,), jnp.float32),
        pltpu.VMEM_SHARED((1, VOCAB), jnp.float32),
        pltpu.SemaphoreType.DMA,
        pltpu.SemaphoreType.DMA,
    ],
    compiler_params=pltpu.CompilerParams(needs_layout_passes=False),
)(_sc_body)


@jax.jit
def kernel(feature_id, feature_val, W, bias):
    fid = feature_id.astype(jnp.int32).T.reshape(-1)
    return _sc_kernel(fid, feature_val.T, W.T, bias)


# speedup vs baseline: 1.0483x; 1.0159x over previous
import functools

import jax
import jax.numpy as jnp
from jax import lax
from jax.experimental import pallas as pl
from jax.experimental.pallas import tpu as pltpu
from jax.experimental.pallas import tpu_sc as plsc

VOCAB = 1000000
BATCH = 16384
FIELDS = 26

NUM_WORKERS = 32
LANES = 16
BPW = BATCH // NUM_WORKERS
IPW = BPW * FIELDS

W_SLICE = 62592
W_LAST = VOCAB - 15 * W_SLICE

GROUPS = ((0, 7), (7, 14), (14, 20))


def _sc_body(fid_hbm, fval_hbm, w_hbm, bias_hbm, out_hbm,
             idx_v, emb_v, val_v, out_v, bias_v, w_sh, sem, wsem, dsem):
    c = lax.axis_index("c")
    s = lax.axis_index("s")
    wid = s * 2 + c
    base_b = wid * BPW

    w_off = s * W_SLICE

    @pl.when(s < 15)
    def _():
        pltpu.make_async_copy(
            w_hbm.at[:, pl.ds(w_off, W_SLICE)],
            w_sh.at[:, pl.ds(w_off, W_SLICE)], wsem).start()

    @pl.when(s == 15)
    def _():
        pltpu.make_async_copy(
            w_hbm.at[:, pl.ds(15 * W_SLICE, W_LAST)],
            w_sh.at[:, pl.ds(15 * W_SLICE, W_LAST)], wsem).start()

    def stage(j, carry):
        pltpu.make_async_copy(
            fid_hbm.at[pl.ds(j * BATCH + base_b, BPW)],
            idx_v.at[pl.ds(j * BPW, BPW)], sem).start()
        return carry

    lax.fori_loop(0, FIELDS, stage, 0)
    col = pl.multiple_of(base_b, BPW)
    for tr, nr in ((0, 8), (8, 8), (16, 8), (24, 2)):
        pltpu.make_async_copy(
            fval_hbm.at[pl.ds(tr, nr), pl.ds(col, BPW)],
            val_v.at[pl.ds(tr, nr), :], sem).start()
    pltpu.sync_copy(bias_hbm, bias_v)

    def drain_idx(j, carry):
        pltpu.make_async_copy(
            fid_hbm.at[pl.ds(j * BATCH + base_b, BPW)],
            idx_v.at[pl.ds(j * BPW, BPW)], sem).wait()
        return carry

    lax.fori_loop(20, FIELDS, drain_idx, 0)

    g_hbm = pltpu.make_async_copy(
        w_hbm.at[0].at[idx_v.at[pl.ds(20 * BPW, 6 * BPW)]],
        emb_v.at[pl.ds(20 * BPW, 6 * BPW)], dsem)
    g_hbm.start()
    lax.fori_loop(0, 20, drain_idx, 0)

    @pl.when(s < 15)
    def _():
        pltpu.make_async_copy(
            w_hbm.at[:, pl.ds(w_off, W_SLICE)],
            w_sh.at[:, pl.ds(w_off, W_SLICE)], wsem).wait()

    @pl.when(s == 15)
    def _():
        pltpu.make_async_copy(
            w_hbm.at[:, pl.ds(15 * W_SLICE, W_LAST)],
            w_sh.at[:, pl.ds(15 * W_SLICE, W_LAST)], wsem).wait()

    plsc.subcore_barrier()

    gathers = []
    for lo, hi in GROUPS:
        n = (hi - lo) * BPW
        g = pltpu.make_async_copy(
            w_sh.at[0].at[idx_v.at[pl.ds(lo * BPW, n)]],
            emb_v.at[pl.ds(lo * BPW, n)], sem)
        g.start()
        gathers.append(g)

    for tr, nr in ((0, 8), (8, 8), (16, 8), (24, 2)):
        pltpu.make_async_copy(
            fval_hbm.at[pl.ds(tr, nr), pl.ds(col, BPW)],
            val_v.at[pl.ds(tr, nr), :], sem).wait()

    zero = jnp.zeros((LANES,), jnp.int32)
    bias = plsc.load_gather(bias_v, [zero])

    for gi, (lo, hi) in enumerate(GROUPS + ((20, FIELDS),)):
        (gathers[gi] if gi < len(GROUPS) else g_hbm).wait()

        def group_body(cc, carry, lo=lo, hi=hi, first=(gi == 0)):
            off = cc * LANES
            acc = bias if first else out_v[pl.ds(off, LANES)]
            for j in range(lo, hi):
                e = emb_v[pl.ds(j * BPW + off, LANES)]
                v = val_v[j, pl.ds(off, LANES)]
                acc = acc + e * v
            out_v[pl.ds(off, LANES)] = acc
            return carry

        lax.fori_loop(0, BPW // LANES, group_body, 0)

    pltpu.sync_copy(out_v, out_hbm.at[pl.ds(base_b, BPW)])


_sc_kernel = functools.partial(
    pl.kernel,
    mesh=plsc.VectorSubcoreMesh(core_axis_name="c", subcore_axis_name="s"),
    out_type=jax.ShapeDtypeStruct((BATCH,), jnp.float32),
    scratch_types=[
        pltpu.VMEM((IPW,), jnp.int32),
        pltpu.VMEM((IPW,), jnp.float32),
        pltpu.VMEM((FIELDS, BPW), jnp.float32),
        pltpu.VMEM((BPW,), jnp.float32),
        pltpu.VMEM((1,), jnp.float32),
        pltpu.VMEM_SHARED((1, VOCAB), jnp.float32),
        pltpu.SemaphoreType.DMA,
        pltpu.SemaphoreType.DMA,
        pltpu.SemaphoreType.DMA,
    ],
    compiler_params=pltpu.CompilerParams(needs_layout_passes=False),
)(_sc_body)


@jax.jit
def kernel(feature_id, feature_val, W, bias):
    fid = feature_id.astype(jnp.int32).T.reshape(-1)
    return _sc_kernel(fid, feature_val.T, W.T, bias)
